# baseline (speedup 1.0000x reference)
import functools
import jax
import jax.numpy as jnp
from jax import lax
from jax.experimental import pallas as pl
from jax.experimental.pallas import tpu as pltpu
from jax.experimental.pallas import tpu_sc as plsc

_D = 64
_SCALE = 8.0

_NC = 2
_NS = 16
_NW = _NC * _NS
_G = 128
_CHUNK = 512


@jax.jit
def _sc_embed(xf, table):
    n_idx_rows, g = xf.shape
    B = n_idx_rows * g
    b_per_w = B // _NW
    n_chunks = b_per_w // _CHUNK
    rows_per_chunk = _CHUNK // _G

    mesh = plsc.VectorSubcoreMesh(
        core_axis_name="c", subcore_axis_name="s", num_cores=_NC
    )

    @functools.partial(
        pl.kernel,
        mesh=mesh,
        out_type=jax.ShapeDtypeStruct((B, _D), jnp.float32),
        scratch_types=[
            pltpu.VMEM((rows_per_chunk, _G), jnp.int32),
            pltpu.VMEM((_CHUNK, _D), jnp.float32),
            pltpu.SemaphoreType.DMA,
        ],
        compiler_params=pltpu.CompilerParams(use_tc_tiling_on_sc=False),
    )
    def k(table_hbm, idx_hbm, out_hbm, idx_v, rows_v, sem):
        wid = lax.axis_index("s") * _NC + lax.axis_index("c")
        idx_base = wid * (b_per_w // _G)
        out_base = wid * b_per_w

        def chunk_body(ch, _):
            pltpu.sync_copy(
                idx_hbm.at[pl.ds(idx_base + ch * rows_per_chunk, rows_per_chunk)],
                idx_v,
            )
            copies = [
                pltpu.async_copy(
                    table_hbm.at[idx_v.at[j]],
                    rows_v.at[pl.ds(j * _G, _G)],
                    sem,
                )
                for j in range(rows_per_chunk)
            ]
            for c in copies:
                c.wait()

            def row_body(i, carry):
                r = rows_v.at[i]
                for t in range(_D // 16):
                    s = pl.ds(t * 16, 16)
                    r[s] = r[s] * _SCALE
                return carry

            lax.fori_loop(0, _CHUNK, row_body, 0)

            pltpu.sync_copy(
                rows_v, out_hbm.at[pl.ds(out_base + ch * _CHUNK, _CHUNK)]
            )
            return _

        lax.fori_loop(0, n_chunks, chunk_body, 0)

    return k(table, xf)


def kernel(x, table):
    B_rows, L = x.shape
    xf = x.reshape(B_rows * L // _G, _G).astype(jnp.int32)
    out = _sc_embed(xf, table)
    return out.reshape(B_rows, L, _D)
